# Initial kernel scaffold; baseline (speedup 1.0000x reference)
import jax
import jax.numpy as jnp
from jax.experimental import pallas as pl


def kernel(points, w1, g1, b1, w2, g2, b2, w3, g3, b3, w4, g4, b4, wf, gf, bf, lw1, lg1, lb1, lw2, lg2, lb2, lw3, lb3):
    raise NotImplementedError("write your pallas kernel here")


# speedup vs baseline: 10.6491x; 10.6491x over previous
import functools

import jax
import jax.numpy as jnp
from jax import lax
from jax.experimental import pallas as pl
from jax.experimental.pallas import tpu as pltpu
from jax.experimental.pallas import tpu_sc as plsc

_B = 4
_N = 2048
_K = 20
_CP = 128
_ROWS = 256
_EPS = 1e-5


def _lrelu(x):
    return jnp.where(x >= 0, x, 0.2 * x)


def _knn_body(x_ref, xt_ref, idx_ref):
    b = pl.program_id(0)
    xr = x_ref[0]
    xt = xt_ref[0]
    d = jnp.dot(xr, xt, preferred_element_type=jnp.float32)
    xx_r = jnp.sum(xr * xr, axis=1, keepdims=True)
    xx_c = jnp.sum(xt * xt, axis=0, keepdims=True)
    pd = 2.0 * d - xx_r - xx_c

    R = xr.shape[0]
    iota = jax.lax.broadcasted_iota(jnp.int32, (R, _N), 1)
    kiota = jax.lax.broadcasted_iota(jnp.int32, (R, _K), 1)
    idx_acc = jnp.zeros((R, _K), jnp.int32)
    neg = jnp.float32(-jnp.inf)
    for t in range(_K):
        m = jnp.max(pd, axis=1, keepdims=True)
        cand = jnp.where(pd == m, iota, _N)
        j = jnp.min(cand, axis=1, keepdims=True)
        idx_acc = jnp.where(kiota == t, j, idx_acc)
        pd = jnp.where(iota == j, neg, pd)
    idx_ref[0] = idx_acc + b * _N


def _knn_topk(x, xt):
    B, N, C = x.shape
    nb = N // _ROWS
    return pl.pallas_call(
        _knn_body,
        grid=(B, nb),
        in_specs=[
            pl.BlockSpec((1, _ROWS, C), lambda b, i: (b, i, 0)),
            pl.BlockSpec((1, C, N), lambda b, i: (b, 0, 0)),
        ],
        out_specs=pl.BlockSpec((1, _ROWS, _K), lambda b, i: (b, i, 0)),
        out_shape=jax.ShapeDtypeStruct((B, N, _K), jnp.int32),
    )(x, xt)


_GCHUNK = 128


def _gather_sc(x_flat, idx_flat):
    M, C = x_flat.shape
    E = idx_flat.shape[0]
    mesh = plsc.VectorSubcoreMesh(core_axis_name="c", subcore_axis_name="s")
    nw = mesh.num_cores * mesh.num_subcores
    per_w = E // nw
    iters = per_w // _GCHUNK

    @functools.partial(
        pl.kernel,
        out_type=jax.ShapeDtypeStruct((E, C), jnp.float32),
        mesh=mesh,
        scratch_types=[
            pltpu.VMEM((_GCHUNK,), jnp.int32),
            pltpu.VMEM((_GCHUNK, C), jnp.float32),
            pltpu.SemaphoreType.DMA,
        ],
    )
    def run(x_hbm, idx_hbm, out_hbm, idx_v, rows_v, sem):
        wid = lax.axis_index("s") * mesh.num_cores + lax.axis_index("c")
        base = wid * per_w

        def body(i, carry):
            rbase = base + i * _GCHUNK
            pltpu.sync_copy(idx_hbm.at[pl.ds(rbase, _GCHUNK)], idx_v)
            pltpu.async_copy(x_hbm.at[idx_v], rows_v, sem).wait()
            pltpu.sync_copy(rows_v, out_hbm.at[pl.ds(rbase, _GCHUNK)])
            return carry

        lax.fori_loop(0, iters, body, 0)

    return run(x_flat, idx_flat)


def _edge_conv_body(g_ref, x_ref, w_ref, s_ref, b_ref, out_ref, *, o_pad):
    xr = x_ref[0]
    g4 = g_ref[0]
    R, C = xr.shape
    g2 = g4.reshape(_K * R, C)
    central = jnp.broadcast_to(xr[None], (_K, R, C)).reshape(_K * R, C)
    diff = g2 - central
    feat = jnp.concatenate([diff, central], axis=1)
    h = jnp.dot(feat, w_ref[...],
                preferred_element_type=jnp.float32)
    h = _lrelu(h * s_ref[...] + b_ref[...])
    mx = jnp.max(h.reshape(_K, R, h.shape[1]), axis=0)
    if o_pad != mx.shape[1]:
        mx = jnp.pad(mx, ((0, 0), (0, o_pad - mx.shape[1])))
    out_ref[0] = mx


def _edge_conv(g, x, wt, s, b, o_pad):
    B, N, C = x.shape
    O = wt.shape[1]
    nb = N // _ROWS
    return pl.pallas_call(
        functools.partial(_edge_conv_body, o_pad=o_pad),
        grid=(B, nb),
        in_specs=[
            pl.BlockSpec((1, _K, _ROWS, C), lambda bb, i: (bb, 0, i, 0)),
            pl.BlockSpec((1, _ROWS, C), lambda bb, i: (bb, i, 0)),
            pl.BlockSpec((2 * C, O), lambda bb, i: (0, 0)),
            pl.BlockSpec((1, O), lambda bb, i: (0, 0)),
            pl.BlockSpec((1, O), lambda bb, i: (0, 0)),
        ],
        out_specs=pl.BlockSpec((1, _ROWS, o_pad), lambda bb, i: (bb, i, 0)),
        out_shape=jax.ShapeDtypeStruct((B, N, o_pad), jnp.float32),
    )(g, x, wt, s, b)


def _global_feat_body(x1_ref, x2_ref, x3_ref, x4_ref, w_ref, s_ref, b_ref,
                      out_ref):
    xc = jnp.concatenate(
        [x1_ref[0][:, :64], x2_ref[0][:, :64], x3_ref[0], x4_ref[0]], axis=1)
    g = jnp.dot(xc, w_ref[...], preferred_element_type=jnp.float32)
    h = _lrelu(g * s_ref[...] + b_ref[...])
    mx = jnp.max(h, axis=0, keepdims=True)
    mn = jnp.sum(h, axis=0, keepdims=True) * (1.0 / _N)
    out_ref[0] = jnp.concatenate([mx, mn], axis=1)


def _global_feat(x1, x2, x3, x4, wt, s, b):
    B, N, _ = x1.shape
    F = wt.shape[1]
    return pl.pallas_call(
        _global_feat_body,
        grid=(B,),
        in_specs=[
            pl.BlockSpec((1, N, x1.shape[2]), lambda bb: (bb, 0, 0)),
            pl.BlockSpec((1, N, x2.shape[2]), lambda bb: (bb, 0, 0)),
            pl.BlockSpec((1, N, x3.shape[2]), lambda bb: (bb, 0, 0)),
            pl.BlockSpec((1, N, x4.shape[2]), lambda bb: (bb, 0, 0)),
            pl.BlockSpec(wt.shape, lambda bb: (0, 0)),
            pl.BlockSpec((1, F), lambda bb: (0, 0)),
            pl.BlockSpec((1, F), lambda bb: (0, 0)),
        ],
        out_specs=pl.BlockSpec((1, 1, 2 * F), lambda bb: (bb, 0, 0)),
        out_shape=jax.ShapeDtypeStruct((B, 1, 2 * F), jnp.float32),
    )(x1, x2, x3, x4, wt, s, b)


def _mlp_body(f_ref, w1_ref, s1_ref, b1_ref, w2_ref, s2_ref, b2_ref,
              w3_ref, b3_ref, out_ref):
    h = jnp.dot(f_ref[...], w1_ref[...], preferred_element_type=jnp.float32)
    h = _lrelu(h * s1_ref[...] + b1_ref[...])
    h = jnp.dot(h, w2_ref[...], preferred_element_type=jnp.float32)
    h = _lrelu(h * s2_ref[...] + b2_ref[...])
    h = jnp.dot(h, w3_ref[...], preferred_element_type=jnp.float32)
    out_ref[...] = h + b3_ref[...]


def _mlp_head(feat, w1, s1, b1, w2, s2, b2, w3, b3):
    return pl.pallas_call(
        _mlp_body,
        out_shape=jax.ShapeDtypeStruct((feat.shape[0], w3.shape[1]),
                                       jnp.float32),
    )(feat, w1, s1, b1, w2, s2, b2, w3, b3)


def kernel(points, w1, g1, b1, w2, g2, b2, w3, g3, b3, w4, g4, b4,
           wf, gf, bf, lw1, lg1, lb1, lw2, lg2, lb2, lw3, lb3):
    den = jnp.sqrt(jnp.float32(1.0 + _EPS))

    x = jnp.pad(points, ((0, 0), (0, 0), (0, _CP - 3)))
    layer_outs = []
    for (W, g, bb, C, O) in ((w1, g1, b1, 3, 64), (w2, g2, b2, 64, 64),
                             (w3, g3, b3, 64, 128), (w4, g4, b4, 128, 256)):
        wt = jnp.zeros((2 * _CP, O), jnp.float32)
        wt = wt.at[:C].set(jnp.transpose(W[:, :C]))
        wt = wt.at[_CP:_CP + C].set(jnp.transpose(W[:, C:]))

        xt = jnp.transpose(x, (0, 2, 1))
        idx = _knn_topk(x, xt)
        idx_t = jnp.transpose(idx, (0, 2, 1))
        gth = _gather_sc(x.reshape(_B * _N, _CP), idx_t.reshape(-1))
        o_pad = max(O, _CP)
        xn = _edge_conv(gth.reshape(_B, _K, _N, _CP), x, wt,
                        (g / den)[None, :], bb[None, :], o_pad)
        layer_outs.append(xn)
        x = xn

    wfT = jnp.transpose(wf)
    feat = _global_feat(layer_outs[0], layer_outs[1], layer_outs[2],
                        layer_outs[3], wfT, (gf / den)[None, :], bf[None, :])
    feat = feat.reshape(_B, 2048)

    out = _mlp_head(feat, jnp.transpose(lw1), (lg1 / den)[None, :],
                    lb1[None, :], jnp.transpose(lw2), (lg2 / den)[None, :],
                    lb2[None, :], jnp.transpose(lw3), lb3[None, :])
    return out


# speedup vs baseline: 12.4960x; 1.1734x over previous
import functools

import jax
import jax.numpy as jnp
from jax import lax
from jax.experimental import pallas as pl
from jax.experimental.pallas import tpu as pltpu
from jax.experimental.pallas import tpu_sc as plsc

_B = 4
_N = 2048
_K = 20
_CP = 128
_ROWS = 256
_EPS = 1e-5


def _lrelu(x):
    return jnp.where(x >= 0, x, 0.2 * x)


def _knn_body(x_ref, xt_ref, idx_ref):
    b = pl.program_id(0)
    xr = x_ref[0]
    xt = xt_ref[0]
    d = jnp.dot(xr, xt, preferred_element_type=jnp.float32)
    xx_r = jnp.sum(xr * xr, axis=1, keepdims=True)
    xx_c = jnp.sum(xt * xt, axis=0, keepdims=True)
    pd = 2.0 * d - xx_r - xx_c

    R = xr.shape[0]
    iota = jax.lax.broadcasted_iota(jnp.int32, (R, _N), 1)
    kiota = jax.lax.broadcasted_iota(jnp.int32, (R, _K), 1)
    idx_acc = jnp.zeros((R, _K), jnp.int32)
    neg = jnp.float32(-jnp.inf)
    for t in range(_K):
        m = jnp.max(pd, axis=1, keepdims=True)
        cand = jnp.where(pd == m, iota, _N)
        j = jnp.min(cand, axis=1, keepdims=True)
        idx_acc = jnp.where(kiota == t, j, idx_acc)
        pd = jnp.where(iota == j, neg, pd)
    idx_ref[0] = idx_acc + b * _N


def _knn_topk(x, xt):
    B, N, C = x.shape
    nb = N // _ROWS
    return pl.pallas_call(
        _knn_body,
        grid=(B, nb),
        in_specs=[
            pl.BlockSpec((1, _ROWS, C), lambda b, i: (b, i, 0)),
            pl.BlockSpec((1, C, N), lambda b, i: (b, 0, 0)),
        ],
        out_specs=pl.BlockSpec((1, _ROWS, _K), lambda b, i: (b, i, 0)),
        out_shape=jax.ShapeDtypeStruct((B, N, _K), jnp.int32),
    )(x, xt)


_GCHUNK = 128


def _gather_sc(x_flat, idx_flat):
    M, C = x_flat.shape
    E = idx_flat.shape[0]
    mesh = plsc.VectorSubcoreMesh(core_axis_name="c", subcore_axis_name="s")
    nw = mesh.num_cores * mesh.num_subcores
    per_w = E // nw
    iters = per_w // _GCHUNK

    @functools.partial(
        pl.kernel,
        out_type=jax.ShapeDtypeStruct((E, C), jnp.float32),
        mesh=mesh,
        scratch_types=[
            pltpu.VMEM((_GCHUNK,), jnp.int32),
            pltpu.VMEM((_GCHUNK, C), jnp.float32),
            pltpu.SemaphoreType.DMA,
        ],
    )
    def run(x_hbm, idx_hbm, out_hbm, idx_v, rows_v, sem):
        wid = lax.axis_index("s") * mesh.num_cores + lax.axis_index("c")
        base = wid * per_w

        def body(i, carry):
            rbase = base + i * _GCHUNK
            pltpu.sync_copy(idx_hbm.at[pl.ds(rbase, _GCHUNK)], idx_v)
            pltpu.async_copy(x_hbm.at[idx_v], rows_v, sem).wait()
            pltpu.sync_copy(rows_v, out_hbm.at[pl.ds(rbase, _GCHUNK)])
            return carry

        lax.fori_loop(0, iters, body, 0)

    return run(x_flat, idx_flat)


def _edge_conv_body(g_ref, x_ref, w_ref, s_ref, b_ref, out_ref, *, o_pad):
    xr = x_ref[0]
    g4 = g_ref[0]
    R, C = xr.shape
    g2 = g4.reshape(_K * R, C)
    central = jnp.broadcast_to(xr[None], (_K, R, C)).reshape(_K * R, C)
    diff = g2 - central
    feat = jnp.concatenate([diff, central], axis=1)
    h = jnp.dot(feat, w_ref[...],
                preferred_element_type=jnp.float32)
    h = _lrelu(h * s_ref[...] + b_ref[...])
    mx = jnp.max(h.reshape(_K, R, h.shape[1]), axis=0)
    if o_pad != mx.shape[1]:
        mx = jnp.pad(mx, ((0, 0), (0, o_pad - mx.shape[1])))
    out_ref[0] = mx


def _edge_conv(g, x, wt, s, b, o_pad):
    B, N, C = x.shape
    O = wt.shape[1]
    nb = N // _ROWS
    return pl.pallas_call(
        functools.partial(_edge_conv_body, o_pad=o_pad),
        grid=(B, nb),
        in_specs=[
            pl.BlockSpec((1, _K, _ROWS, C), lambda bb, i: (bb, 0, i, 0)),
            pl.BlockSpec((1, _ROWS, C), lambda bb, i: (bb, i, 0)),
            pl.BlockSpec((2 * C, O), lambda bb, i: (0, 0)),
            pl.BlockSpec((1, O), lambda bb, i: (0, 0)),
            pl.BlockSpec((1, O), lambda bb, i: (0, 0)),
        ],
        out_specs=pl.BlockSpec((1, _ROWS, o_pad), lambda bb, i: (bb, i, 0)),
        out_shape=jax.ShapeDtypeStruct((B, N, o_pad), jnp.float32),
    )(g, x, wt, s, b)


def _global_feat_body(x1_ref, x2_ref, x3_ref, x4_ref, w_ref, s_ref, b_ref,
                      out_ref):
    xc = jnp.concatenate(
        [x1_ref[0][:, :64], x2_ref[0][:, :64], x3_ref[0], x4_ref[0]], axis=1)
    g = jnp.dot(xc, w_ref[...], preferred_element_type=jnp.float32)
    h = _lrelu(g * s_ref[...] + b_ref[...])
    mx = jnp.max(h, axis=0, keepdims=True)
    mn = jnp.sum(h, axis=0, keepdims=True) * (1.0 / _N)
    out_ref[0] = jnp.concatenate([mx, mn], axis=1)


def _global_feat(x1, x2, x3, x4, wt, s, b):
    B, N, _ = x1.shape
    F = wt.shape[1]
    return pl.pallas_call(
        _global_feat_body,
        grid=(B,),
        in_specs=[
            pl.BlockSpec((1, N, x1.shape[2]), lambda bb: (bb, 0, 0)),
            pl.BlockSpec((1, N, x2.shape[2]), lambda bb: (bb, 0, 0)),
            pl.BlockSpec((1, N, x3.shape[2]), lambda bb: (bb, 0, 0)),
            pl.BlockSpec((1, N, x4.shape[2]), lambda bb: (bb, 0, 0)),
            pl.BlockSpec(wt.shape, lambda bb: (0, 0)),
            pl.BlockSpec((1, F), lambda bb: (0, 0)),
            pl.BlockSpec((1, F), lambda bb: (0, 0)),
        ],
        out_specs=pl.BlockSpec((1, 1, 2 * F), lambda bb: (bb, 0, 0)),
        out_shape=jax.ShapeDtypeStruct((B, 1, 2 * F), jnp.float32),
    )(x1, x2, x3, x4, wt, s, b)


def _mlp_body(f_ref, w1_ref, s1_ref, b1_ref, w2_ref, s2_ref, b2_ref,
              w3_ref, b3_ref, out_ref):
    h = jnp.dot(f_ref[...], w1_ref[...], preferred_element_type=jnp.float32)
    h = _lrelu(h * s1_ref[...] + b1_ref[...])
    h = jnp.dot(h, w2_ref[...], preferred_element_type=jnp.float32)
    h = _lrelu(h * s2_ref[...] + b2_ref[...])
    h = jnp.dot(h, w3_ref[...], preferred_element_type=jnp.float32)
    out_ref[...] = h + b3_ref[...]


def _mlp_head(feat, w1, s1, b1, w2, s2, b2, w3, b3):
    return pl.pallas_call(
        _mlp_body,
        out_shape=jax.ShapeDtypeStruct((feat.shape[0], w3.shape[1]),
                                       jnp.float32),
    )(feat, w1, s1, b1, w2, s2, b2, w3, b3)


def kernel(points, w1, g1, b1, w2, g2, b2, w3, g3, b3, w4, g4, b4,
           wf, gf, bf, lw1, lg1, lb1, lw2, lg2, lb2, lw3, lb3):
    den = jnp.sqrt(jnp.float32(1.0 + _EPS))

    xp = jnp.pad(points, ((0, 0), (0, 0), (0, _CP - 3)))
    halves = [xp[:2], xp[2:]]
    layer_outs = []
    for (W, g, bb, C, O) in ((w1, g1, b1, 3, 64), (w2, g2, b2, 64, 64),
                             (w3, g3, b3, 64, 128), (w4, g4, b4, 128, 256)):
        wt = jnp.zeros((2 * _CP, O), jnp.float32)
        wt = wt.at[:C].set(jnp.transpose(W[:, :C]))
        wt = wt.at[_CP:_CP + C].set(jnp.transpose(W[:, C:]))
        s = (g / den)[None, :]
        bb2 = bb[None, :]
        o_pad = max(O, _CP)

        new_halves = []
        for xh in halves:
            hb = xh.shape[0]
            xt = jnp.transpose(xh, (0, 2, 1))
            idx = _knn_topk(xh, xt)
            idx_t = jnp.transpose(idx, (0, 2, 1))
            gth = _gather_sc(xh.reshape(hb * _N, _CP), idx_t.reshape(-1))
            xn = _edge_conv(gth.reshape(hb, _K, _N, _CP), xh, wt, s, bb2,
                            o_pad)
            new_halves.append(xn)
        halves = new_halves
        layer_outs.append(halves)

    wfT = jnp.transpose(wf)
    sf = (gf / den)[None, :]
    bf2 = bf[None, :]
    feat = jnp.concatenate(
        [_global_feat(layer_outs[0][h], layer_outs[1][h], layer_outs[2][h],
                      layer_outs[3][h], wfT, sf, bf2) for h in (0, 1)],
        axis=0)
    feat = feat.reshape(_B, 2048)

    out = _mlp_head(feat, jnp.transpose(lw1), (lg1 / den)[None, :],
                    lb1[None, :], jnp.transpose(lw2), (lg2 / den)[None, :],
                    lb2[None, :], jnp.transpose(lw3), lb3[None, :])
    return out


# speedup vs baseline: 13.8261x; 1.1064x over previous
import functools

import jax
import jax.numpy as jnp
from jax import lax
from jax.experimental import pallas as pl
from jax.experimental.pallas import tpu as pltpu
from jax.experimental.pallas import tpu_sc as plsc

_B = 4
_N = 2048
_K = 20
_CP = 128
_ROWS = 512
_EPS = 1e-5


def _lrelu(x):
    return jnp.where(x >= 0, x, 0.2 * x)


def _knn_body(x_ref, xt_ref, idx_ref):
    b = pl.program_id(0)
    xr = x_ref[0]
    xt = xt_ref[0]
    d = jnp.dot(xr, xt, preferred_element_type=jnp.float32)
    xx_r = jnp.sum(xr * xr, axis=1, keepdims=True)
    xx_c = jnp.sum(xt * xt, axis=0, keepdims=True)
    pd = 2.0 * d - xx_r - xx_c

    R = xr.shape[0]
    iota = jax.lax.broadcasted_iota(jnp.int32, (R, _N), 1)
    kiota = jax.lax.broadcasted_iota(jnp.int32, (R, _K), 1)
    idx_acc = jnp.zeros((R, _K), jnp.int32)
    neg = jnp.float32(-jnp.inf)
    for t in range(_K):
        m = jnp.max(pd, axis=1, keepdims=True)
        cand = jnp.where(pd == m, iota, _N)
        j = jnp.min(cand, axis=1, keepdims=True)
        idx_acc = jnp.where(kiota == t, j, idx_acc)
        pd = jnp.where(iota == j, neg, pd)
    idx_ref[0] = idx_acc + b * _N


def _knn_topk(x, xt):
    B, N, C = x.shape
    nb = N // _ROWS
    return pl.pallas_call(
        _knn_body,
        grid=(B, nb),
        in_specs=[
            pl.BlockSpec((1, _ROWS, C), lambda b, i: (b, i, 0)),
            pl.BlockSpec((1, C, N), lambda b, i: (b, 0, 0)),
        ],
        out_specs=pl.BlockSpec((1, _ROWS, _K), lambda b, i: (b, i, 0)),
        out_shape=jax.ShapeDtypeStruct((B, N, _K), jnp.int32),
    )(x, xt)


_GCHUNK = 128


def _gather_sc(x_flat, idx_flat):
    M, C = x_flat.shape
    E = idx_flat.shape[0]
    mesh = plsc.VectorSubcoreMesh(core_axis_name="c", subcore_axis_name="s")
    nw = mesh.num_cores * mesh.num_subcores
    per_w = E // nw
    iters = per_w // _GCHUNK

    @functools.partial(
        pl.kernel,
        out_type=jax.ShapeDtypeStruct((E, C), jnp.float32),
        mesh=mesh,
        scratch_types=[
            pltpu.VMEM((_GCHUNK,), jnp.int32),
            pltpu.VMEM((_GCHUNK, C), jnp.float32),
            pltpu.SemaphoreType.DMA,
        ],
    )
    def run(x_hbm, idx_hbm, out_hbm, idx_v, rows_v, sem):
        wid = lax.axis_index("s") * mesh.num_cores + lax.axis_index("c")
        base = wid * per_w

        def body(i, carry):
            rbase = base + i * _GCHUNK
            pltpu.sync_copy(idx_hbm.at[pl.ds(rbase, _GCHUNK)], idx_v)
            pltpu.async_copy(x_hbm.at[idx_v], rows_v, sem).wait()
            pltpu.sync_copy(rows_v, out_hbm.at[pl.ds(rbase, _GCHUNK)])
            return carry

        lax.fori_loop(0, iters, body, 0)

    return run(x_flat, idx_flat)


def _edge_conv_body(g_ref, x_ref, w_ref, s_ref, b_ref, out_ref, *, o_pad):
    xr = x_ref[0]
    g4 = g_ref[0]
    R, C = xr.shape
    g2 = g4.reshape(_K * R, C)
    central = jnp.broadcast_to(xr[None], (_K, R, C)).reshape(_K * R, C)
    diff = g2 - central
    feat = jnp.concatenate([diff, central], axis=1)
    h = jnp.dot(feat, w_ref[...],
                preferred_element_type=jnp.float32)
    h = _lrelu(h * s_ref[...] + b_ref[...])
    mx = jnp.max(h.reshape(_K, R, h.shape[1]), axis=0)
    if o_pad != mx.shape[1]:
        mx = jnp.pad(mx, ((0, 0), (0, o_pad - mx.shape[1])))
    out_ref[0] = mx


def _edge_conv(g, x, wt, s, b, o_pad):
    B, N, C = x.shape
    O = wt.shape[1]
    nb = N // _ROWS
    return pl.pallas_call(
        functools.partial(_edge_conv_body, o_pad=o_pad),
        grid=(B, nb),
        in_specs=[
            pl.BlockSpec((1, _K, _ROWS, C), lambda bb, i: (bb, 0, i, 0)),
            pl.BlockSpec((1, _ROWS, C), lambda bb, i: (bb, i, 0)),
            pl.BlockSpec((2 * C, O), lambda bb, i: (0, 0)),
            pl.BlockSpec((1, O), lambda bb, i: (0, 0)),
            pl.BlockSpec((1, O), lambda bb, i: (0, 0)),
        ],
        out_specs=pl.BlockSpec((1, _ROWS, o_pad), lambda bb, i: (bb, i, 0)),
        out_shape=jax.ShapeDtypeStruct((B, N, o_pad), jnp.float32),
    )(g, x, wt, s, b)


def _global_feat_body(x1_ref, x2_ref, x3_ref, x4_ref, w_ref, s_ref, b_ref,
                      out_ref):
    xc = jnp.concatenate(
        [x1_ref[0][:, :64], x2_ref[0][:, :64], x3_ref[0], x4_ref[0]], axis=1)
    g = jnp.dot(xc, w_ref[...], preferred_element_type=jnp.float32)
    h = _lrelu(g * s_ref[...] + b_ref[...])
    mx = jnp.max(h, axis=0, keepdims=True)
    mn = jnp.sum(h, axis=0, keepdims=True) * (1.0 / _N)
    out_ref[0] = jnp.concatenate([mx, mn], axis=1)


def _global_feat(x1, x2, x3, x4, wt, s, b):
    B, N, _ = x1.shape
    F = wt.shape[1]
    return pl.pallas_call(
        _global_feat_body,
        grid=(B,),
        in_specs=[
            pl.BlockSpec((1, N, x1.shape[2]), lambda bb: (bb, 0, 0)),
            pl.BlockSpec((1, N, x2.shape[2]), lambda bb: (bb, 0, 0)),
            pl.BlockSpec((1, N, x3.shape[2]), lambda bb: (bb, 0, 0)),
            pl.BlockSpec((1, N, x4.shape[2]), lambda bb: (bb, 0, 0)),
            pl.BlockSpec(wt.shape, lambda bb: (0, 0)),
            pl.BlockSpec((1, F), lambda bb: (0, 0)),
            pl.BlockSpec((1, F), lambda bb: (0, 0)),
        ],
        out_specs=pl.BlockSpec((1, 1, 2 * F), lambda bb: (bb, 0, 0)),
        out_shape=jax.ShapeDtypeStruct((B, 1, 2 * F), jnp.float32),
    )(x1, x2, x3, x4, wt, s, b)


def _mlp_body(f_ref, w1_ref, s1_ref, b1_ref, w2_ref, s2_ref, b2_ref,
              w3_ref, b3_ref, out_ref):
    h = jnp.dot(f_ref[...], w1_ref[...], preferred_element_type=jnp.float32)
    h = _lrelu(h * s1_ref[...] + b1_ref[...])
    h = jnp.dot(h, w2_ref[...], preferred_element_type=jnp.float32)
    h = _lrelu(h * s2_ref[...] + b2_ref[...])
    h = jnp.dot(h, w3_ref[...], preferred_element_type=jnp.float32)
    out_ref[...] = h + b3_ref[...]


def _mlp_head(feat, w1, s1, b1, w2, s2, b2, w3, b3):
    return pl.pallas_call(
        _mlp_body,
        out_shape=jax.ShapeDtypeStruct((feat.shape[0], w3.shape[1]),
                                       jnp.float32),
    )(feat, w1, s1, b1, w2, s2, b2, w3, b3)


def kernel(points, w1, g1, b1, w2, g2, b2, w3, g3, b3, w4, g4, b4,
           wf, gf, bf, lw1, lg1, lb1, lw2, lg2, lb2, lw3, lb3):
    den = jnp.sqrt(jnp.float32(1.0 + _EPS))

    xp = jnp.pad(points, ((0, 0), (0, 0), (0, _CP - 3)))
    halves = [xp[:2], xp[2:]]
    layer_outs = []
    for (W, g, bb, C, O) in ((w1, g1, b1, 3, 64), (w2, g2, b2, 64, 64),
                             (w3, g3, b3, 64, 128), (w4, g4, b4, 128, 256)):
        wt = jnp.zeros((2 * _CP, O), jnp.float32)
        wt = wt.at[:C].set(jnp.transpose(W[:, :C]))
        wt = wt.at[_CP:_CP + C].set(jnp.transpose(W[:, C:]))
        s = (g / den)[None, :]
        bb2 = bb[None, :]
        o_pad = max(O, _CP)

        new_halves = []
        for xh in halves:
            hb = xh.shape[0]
            xt = jnp.transpose(xh, (0, 2, 1))
            idx = _knn_topk(xh, xt)
            idx_t = jnp.transpose(idx, (0, 2, 1))
            gth = _gather_sc(xh.reshape(hb * _N, _CP), idx_t.reshape(-1))
            xn = _edge_conv(gth.reshape(hb, _K, _N, _CP), xh, wt, s, bb2,
                            o_pad)
            new_halves.append(xn)
        halves = new_halves
        layer_outs.append(halves)

    wfT = jnp.transpose(wf)
    sf = (gf / den)[None, :]
    bf2 = bf[None, :]
    feat = jnp.concatenate(
        [_global_feat(layer_outs[0][h], layer_outs[1][h], layer_outs[2][h],
                      layer_outs[3][h], wfT, sf, bf2) for h in (0, 1)],
        axis=0)
    feat = feat.reshape(_B, 2048)

    out = _mlp_head(feat, jnp.transpose(lw1), (lg1 / den)[None, :],
                    lb1[None, :], jnp.transpose(lw2), (lg2 / den)[None, :],
                    lb2[None, :], jnp.transpose(lw3), lb3[None, :])
    return out
